# baseline (speedup 1.0000x reference)
import jax
import jax.numpy as jnp
from jax.experimental import pallas as pl
from jax.experimental.pallas import tpu as pltpu

_BLK = 256


def _fused_body(x_ref, a_ref, wself_ref, bself_ref, wnb_ref, bnb_ref, watt_ref,
                gamma_ref, beta_ref, out_ref, m_scr):
    i = pl.program_id(0)
    B, N, F = x_ref.shape

    @pl.when(i == 0)
    def _prep():
        w2 = watt_ref[1:2, :]
        es = []
        for b in range(B):
            x = x_ref[b]
            nb = (jnp.dot(x, wnb_ref[...], preferred_element_type=jnp.float32)
                  + bnb_ref[...])
            att = jnp.sum(x * w2, axis=1, keepdims=True)
            e = jnp.exp(att - jnp.max(att))
            m_scr[:, b * F:(b + 1) * F] = e * nb
            es.append(e)
        es.append(jnp.zeros((N, F - B), dtype=jnp.float32))
        m_scr[:, B * F:] = jnp.concatenate(es, axis=1)

    mm = jnp.dot(a_ref[...], m_scr[...], preferred_element_type=jnp.float32)
    for b in range(B):
        x_blk = x_ref[b, pl.ds(i * _BLK, _BLK), :]
        self_blk = (jnp.dot(x_blk, wself_ref[...],
                            preferred_element_type=jnp.float32)
                    + bself_ref[...])
        num = mm[:, b * F:(b + 1) * F]
        den = mm[:, B * F + b:B * F + b + 1]
        rec = jnp.where(den > 0, 1.0 / den, 0.0)
        comb = self_blk + num * rec
        mean = jnp.mean(comb, axis=1, keepdims=True)
        cent = comb - mean
        var = jnp.mean(cent * cent, axis=1, keepdims=True)
        rstd = jax.lax.rsqrt(var + 1e-5)
        out_ref[b] = jnp.maximum(
            (cent * rstd) * gamma_ref[...] + beta_ref[...], 0.0)


def kernel(node_features, adjacency_matrix, W_self, b_self, W_nb, b_nb,
           W_att, b_att, ln_gamma, ln_beta):
    B, N, F = node_features.shape
    watt2 = W_att.reshape(2, F)
    bself = b_self.reshape(1, F)
    bnb = b_nb.reshape(1, F)
    gamma = ln_gamma.reshape(1, F)
    beta = ln_beta.reshape(1, F)

    grid = (N // _BLK,)
    out = pl.pallas_call(
        _fused_body,
        grid=grid,
        in_specs=[
            pl.BlockSpec((B, N, F), lambda i: (0, 0, 0)),
            pl.BlockSpec((_BLK, N), lambda i: (i, 0)),
            pl.BlockSpec((F, F), lambda i: (0, 0)),
            pl.BlockSpec((1, F), lambda i: (0, 0)),
            pl.BlockSpec((F, F), lambda i: (0, 0)),
            pl.BlockSpec((1, F), lambda i: (0, 0)),
            pl.BlockSpec((2, F), lambda i: (0, 0)),
            pl.BlockSpec((1, F), lambda i: (0, 0)),
            pl.BlockSpec((1, F), lambda i: (0, 0)),
        ],
        out_specs=pl.BlockSpec((B, _BLK, F), lambda i: (0, i, 0)),
        out_shape=jax.ShapeDtypeStruct((B, N, F), jnp.float32),
        scratch_shapes=[
            pltpu.VMEM((N, (B + 1) * F), jnp.float32),
        ],
        compiler_params=pltpu.CompilerParams(
            dimension_semantics=("arbitrary",),
        ),
    )(node_features, adjacency_matrix, W_self, bself, W_nb, bnb, watt2,
      gamma, beta)
    return out


# speedup vs baseline: 1.0325x; 1.0325x over previous
import jax
import jax.numpy as jnp
from jax.experimental import pallas as pl
from jax.experimental.pallas import tpu as pltpu

_BLK = 256


def _fused_body(x_ref, a_ref, wself_ref, bself_ref, wnb_ref, bnb_ref, watt_ref,
                gamma_ref, beta_ref, out_ref, m_scr):
    i = pl.program_id(0)
    B, N, F = x_ref.shape

    @pl.when(i == 0)
    def _prep():
        w2 = watt_ref[1:2, :]
        es = []
        for b in range(B):
            x = x_ref[b]
            nb = (jnp.dot(x, wnb_ref[...], preferred_element_type=jnp.float32)
                  + bnb_ref[...])
            att = jnp.sum(x * w2, axis=1, keepdims=True)
            e = jnp.exp(att - jnp.max(att))
            m_scr[:, b * F:(b + 1) * F] = (e * nb).astype(m_scr.dtype)
            es.append(e)
        es.append(jnp.zeros((N, F - B), dtype=jnp.float32))
        m_scr[:, B * F:] = jnp.concatenate(es, axis=1).astype(m_scr.dtype)

    mm = jnp.dot(a_ref[...].astype(m_scr.dtype), m_scr[...],
                 preferred_element_type=jnp.float32)
    for b in range(B):
        x_blk = x_ref[b, pl.ds(i * _BLK, _BLK), :]
        self_blk = (jnp.dot(x_blk, wself_ref[...],
                            preferred_element_type=jnp.float32)
                    + bself_ref[...])
        num = mm[:, b * F:(b + 1) * F]
        den = mm[:, B * F + b:B * F + b + 1]
        rec = jnp.where(den > 0, 1.0 / den, 0.0)
        comb = self_blk + num * rec
        mean = jnp.mean(comb, axis=1, keepdims=True)
        cent = comb - mean
        var = jnp.mean(cent * cent, axis=1, keepdims=True)
        rstd = jax.lax.rsqrt(var + 1e-5)
        out_ref[b] = jnp.maximum(
            (cent * rstd) * gamma_ref[...] + beta_ref[...], 0.0)


def kernel(node_features, adjacency_matrix, W_self, b_self, W_nb, b_nb,
           W_att, b_att, ln_gamma, ln_beta):
    B, N, F = node_features.shape
    watt2 = W_att.reshape(2, F)
    bself = b_self.reshape(1, F)
    bnb = b_nb.reshape(1, F)
    gamma = ln_gamma.reshape(1, F)
    beta = ln_beta.reshape(1, F)

    grid = (N // _BLK,)
    out = pl.pallas_call(
        _fused_body,
        grid=grid,
        in_specs=[
            pl.BlockSpec((B, N, F), lambda i: (0, 0, 0)),
            pl.BlockSpec((_BLK, N), lambda i: (0, 0)),
            pl.BlockSpec((F, F), lambda i: (0, 0)),
            pl.BlockSpec((1, F), lambda i: (0, 0)),
            pl.BlockSpec((F, F), lambda i: (0, 0)),
            pl.BlockSpec((1, F), lambda i: (0, 0)),
            pl.BlockSpec((2, F), lambda i: (0, 0)),
            pl.BlockSpec((1, F), lambda i: (0, 0)),
            pl.BlockSpec((1, F), lambda i: (0, 0)),
        ],
        out_specs=pl.BlockSpec((B, _BLK, F), lambda i: (0, i, 0)),
        out_shape=jax.ShapeDtypeStruct((B, N, F), jnp.float32),
        scratch_shapes=[
            pltpu.VMEM((N, (B + 1) * F), jnp.bfloat16),
        ],
        compiler_params=pltpu.CompilerParams(
            dimension_semantics=("arbitrary",),
        ),
    )(node_features, adjacency_matrix, W_self, bself, W_nb, bnb, watt2,
      gamma, beta)
    return out
